# baseline (speedup 1.0000x reference)
import functools

import jax
import jax.numpy as jnp
from jax import lax
from jax.experimental import pallas as pl
from jax.experimental.pallas import tpu as pltpu
from jax.experimental.pallas import tpu_sc as plsc

D = 128
NC = 2
NS = 16
NW = NC * NS
CHUNK = 400
LANES = 16


def _transform_table(h, w):
    rows, d = h.shape
    blk = 2000
    assert rows % blk == 0

    def body(x_ref, w_ref, o_ref):
        o_ref[...] = lax.dot_general(
            x_ref[...], w_ref[...],
            dimension_numbers=(((1,), (1,)), ((), ())),
            preferred_element_type=jnp.float32)

    return pl.pallas_call(
        body,
        grid=(rows // blk,),
        in_specs=[
            pl.BlockSpec((blk, d), lambda i: (i, 0)),
            pl.BlockSpec((d, d), lambda i: (0, 0)),
        ],
        out_specs=pl.BlockSpec((blk, d), lambda i: (i, 0)),
        out_shape=jax.ShapeDtypeStruct((rows, d), jnp.float32),
    )(h, w)


def _pack_words(table):
    rows = table.shape[0]
    v = table.astype(jnp.bfloat16).reshape(rows // 2, 2, D)
    lo = lax.convert_element_type(
        lax.bitcast_convert_type(v[:, :, :DW], jnp.uint16), jnp.uint32)
    hi = lax.convert_element_type(
        lax.bitcast_convert_type(v[:, :, DW:], jnp.uint16), jnp.uint32)
    w3 = lo | (hi << jnp.uint32(16))
    return lax.bitcast_convert_type(w3, jnp.int32).reshape(rows // 2, D)


DW = D // 2


def _make_sc_packer(rows_total):
    per_w = rows_total // NW
    rch = 125
    n_chunks = per_w // rch
    mesh = plsc.VectorSubcoreMesh(core_axis_name="c", subcore_axis_name="s")

    @functools.partial(
        pl.kernel,
        mesh=mesh,
        compiler_params=pltpu.CompilerParams(
            needs_layout_passes=False, use_tc_tiling_on_sc=False),
        out_type=[jax.ShapeDtypeStruct((rows_total, DW), jnp.int32),
                  jax.ShapeDtypeStruct((rows_total, DW), jnp.int32)],
        scratch_types=[
            pltpu.VMEM((rch * D,), jnp.float32),
            pltpu.VMEM((rch * D,), jnp.float32),
            pltpu.VMEM((rch * D,), jnp.float32),
            pltpu.VMEM((rch * D,), jnp.float32),
            pltpu.VMEM((rch, DW), jnp.int32),
            pltpu.VMEM((rch, DW), jnp.int32),
            pltpu.VMEM((rch, DW), jnp.int32),
            pltpu.VMEM((rch, DW), jnp.int32),
            pltpu.SemaphoreType.DMA,
            pltpu.SemaphoreType.DMA,
            pltpu.SemaphoreType.DMA,
            pltpu.SemaphoreType.DMA,
            pltpu.SemaphoreType.DMA,
            pltpu.SemaphoreType.DMA,
            pltpu.SemaphoreType.DMA,
            pltpu.SemaphoreType.DMA,
        ],
    )
    def packer(hu_flat, hi_flat, hu_out, hi_out,
               hu_in_a, hu_in_b, hi_in_a, hi_in_b,
               hu_pk_a, hu_pk_b, hi_pk_a, hi_pk_b,
               s_hu_a, s_hu_b, s_hi_a, s_hi_b,
               so_hu_a, so_hu_b, so_hi_a, so_hi_b):
        wid = lax.axis_index("s") * NC + lax.axis_index("c")
        base_row = wid * per_w

        def fire_in(tab, c, buf, sem):
            off = (base_row + c * rch) * D
            pltpu.async_copy(tab.at[pl.ds(off, rch * D)], buf, sem)

        def drain_in(tab, buf, sem):
            pltpu.make_async_copy(tab.at[pl.ds(0, rch * D)], buf, sem).wait()

        def fire_out(out_hbm, c, pk, sem):
            pltpu.async_copy(pk, out_hbm.at[pl.ds(base_row + c * rch, rch)], sem)

        def drain_out(out_hbm, pk, sem):
            pltpu.make_async_copy(pk, out_hbm.at[pl.ds(base_row, rch)], sem).wait()

        def compute(buf, pk):
            def row_body(r, _):
                for k in range(DW // LANES):
                    a = buf[pl.ds(r * D + 2 * k * LANES, LANES)]
                    b = buf[pl.ds(r * D + (2 * k + 1) * LANES, LANES)]
                    pk[r, pl.ds(k * LANES, LANES)] = plsc.bitcast(
                        plsc.pack(a, b, format=plsc.PackFormat.INTERLEAVED),
                        jnp.int32)
                return 0
            lax.fori_loop(0, rch, row_body, 0)

        def step(tab, out_hbm, c, in_buf, in_sem, nxt_buf, nxt_sem,
                 pk, pk_sem):
            drain_in(tab, in_buf, in_sem)
            fire_in(tab, c + 1, nxt_buf, nxt_sem)
            drain_out(out_hbm, pk, pk_sem)
            compute(in_buf, pk)
            fire_out(out_hbm, c, pk, pk_sem)

        fire_in(hu_flat, 0, hu_in_a, s_hu_a)
        fire_in(hi_flat, 0, hi_in_a, s_hi_a)
        fire_out(hu_out, 0, hu_pk_a, so_hu_a)
        fire_out(hu_out, 1, hu_pk_b, so_hu_b)
        fire_out(hi_out, 0, hi_pk_a, so_hi_a)
        fire_out(hi_out, 1, hi_pk_b, so_hi_b)

        def pair_body(p, _):
            c0 = 2 * p
            step(hu_flat, hu_out, c0, hu_in_a, s_hu_a, hu_in_b, s_hu_b,
                 hu_pk_a, so_hu_a)
            step(hi_flat, hi_out, c0, hi_in_a, s_hi_a, hi_in_b, s_hi_b,
                 hi_pk_a, so_hi_a)
            step(hu_flat, hu_out, c0 + 1, hu_in_b, s_hu_b, hu_in_a, s_hu_a,
                 hu_pk_b, so_hu_b)
            step(hi_flat, hi_out, c0 + 1, hi_in_b, s_hi_b, hi_in_a, s_hi_a,
                 hi_pk_b, so_hi_b)
            return 0

        lax.fori_loop(0, (n_chunks - 1) // 2, pair_body, 0)

        for tab, out_hbm, in_buf, in_sem, pk, pk_sem in (
                (hu_flat, hu_out, hu_in_a, s_hu_a, hu_pk_a, so_hu_a),
                (hi_flat, hi_out, hi_in_a, s_hi_a, hi_pk_a, so_hi_a)):
            drain_in(tab, in_buf, in_sem)
            drain_out(out_hbm, pk, pk_sem)
            compute(in_buf, pk)
            fire_out(out_hbm, n_chunks - 1, pk, pk_sem)
            drain_out(out_hbm, pk, pk_sem)
        drain_out(hu_out, hu_pk_b, so_hu_b)
        drain_out(hi_out, hi_pk_b, so_hi_b)

    return packer


def _make_sc_scorer(e_total):
    per_w = e_total // NW
    n_chunks = per_w // CHUNK
    groups = CHUNK // LANES
    mesh = plsc.VectorSubcoreMesh(core_axis_name="c", subcore_axis_name="s")

    @functools.partial(
        pl.kernel,
        mesh=mesh,
        compiler_params=pltpu.CompilerParams(
            needs_layout_passes=False, use_tc_tiling_on_sc=False),
        out_type=jax.ShapeDtypeStruct((e_total,), jnp.float32),
        scratch_types=[
            pltpu.VMEM((per_w,), jnp.int32),
            pltpu.VMEM((per_w,), jnp.int32),
            pltpu.VMEM((CHUNK,), jnp.float32),
            pltpu.VMEM((CHUNK, DW), jnp.int32),
            pltpu.VMEM((CHUNK, DW), jnp.int32),
            pltpu.VMEM((CHUNK, DW), jnp.int32),
            pltpu.VMEM((CHUNK, DW), jnp.int32),
            pltpu.SemaphoreType.DMA,
            pltpu.SemaphoreType.DMA,
        ],
    )
    def scorer(hu_t, hi, src_hbm, dst_hbm, out_hbm,
               sidx_v, didx_v, out_v, srows_a, drows_a, srows_b, drows_b,
               sem_a, sem_b):
        wid = lax.axis_index("s") * NC + lax.axis_index("c")
        base = wid * per_w
        pltpu.sync_copy(src_hbm.at[pl.ds(base, per_w)], sidx_v)
        pltpu.sync_copy(dst_hbm.at[pl.ds(base, per_w)], didx_v)

        def fire(c, s_buf, d_buf, sem):
            off = c * CHUNK
            pltpu.async_copy(hu_t.at[sidx_v.at[pl.ds(off, CHUNK)]], s_buf, sem)
            pltpu.async_copy(hi.at[didx_v.at[pl.ds(off, CHUNK)]], d_buf, sem)

        def drain(s_buf, d_buf, sem):
            pltpu.make_async_copy(hu_t.at[sidx_v.at[pl.ds(0, CHUNK)]], s_buf, sem).wait()
            pltpu.make_async_copy(hi.at[didx_v.at[pl.ds(0, CHUNK)]], d_buf, sem).wait()

        lane_iota = lax.iota(jnp.int32, LANES)

        def compute_chunk(c, s_ref, d_ref):
            def group_body(g, _):
                res = jnp.zeros((LANES,), jnp.float32)
                for j in range(LANES):
                    accs = []
                    for k in range(DW // LANES):
                        sw = s_ref[g * LANES + j, pl.ds(k * LANES, LANES)]
                        dw = d_ref[g * LANES + j, pl.ds(k * LANES, LANES)]
                        prod = (plsc.bitcast(sw, jnp.bfloat16)
                                * plsc.bitcast(dw, jnp.bfloat16))
                        p0, p1 = plsc.unpack(
                            prod, format=plsc.PackFormat.INTERLEAVED)
                        accs.append(p0 + p1)
                    acc = (accs[0] + accs[1]) + (accs[2] + accs[3])
                    res = jnp.where(lane_iota == j, jnp.sum(acc), res)
                out_v[pl.ds(g * LANES, LANES)] = res
                return 0
            lax.fori_loop(0, groups, group_body, 0)
            pltpu.sync_copy(out_v, out_hbm.at[pl.ds(base + c * CHUNK, CHUNK)])

        assert n_chunks % 2 == 1
        fire(0, srows_a, drows_a, sem_a)

        def pair_body(p, _):
            c0 = 2 * p
            drain(srows_a, drows_a, sem_a)
            fire(c0 + 1, srows_b, drows_b, sem_b)
            compute_chunk(c0, srows_a, drows_a)
            drain(srows_b, drows_b, sem_b)
            fire(c0 + 2, srows_a, drows_a, sem_a)
            compute_chunk(c0 + 1, srows_b, drows_b)
            return 0

        lax.fori_loop(0, (n_chunks - 1) // 2, pair_body, 0)
        drain(srows_a, drows_a, sem_a)
        compute_chunk(n_chunks - 1, srows_a, drows_a)

    return scorer


def kernel(h_user, h_item, W, src_idx, dst_idx):
    rows = h_user.shape[0]
    hu_p2 = _pack_words(_transform_table(h_user, W))
    hi_p2 = _pack_words(h_item)
    scorer = _make_sc_scorer(src_idx.shape[0])
    return scorer(hu_p2.reshape(rows, DW), hi_p2.reshape(rows, DW),
                  src_idx, dst_idx)


# speedup vs baseline: 1.8118x; 1.8118x over previous
import functools

import jax
import jax.numpy as jnp
from jax import lax
from jax.experimental import pallas as pl
from jax.experimental.pallas import tpu as pltpu
from jax.experimental.pallas import tpu_sc as plsc

D = 128
NC = 2
NS = 16
NW = NC * NS
CHUNK = 400
LANES = 16


def _pack_words(y):
    b = lax.bitcast_convert_type(y, jnp.uint32) + jnp.uint32(0x8000)
    lo = b[:, :DW] >> jnp.uint32(16)
    hi = b[:, DW:] & jnp.uint32(0xFFFF0000)
    return lax.bitcast_convert_type(lo | hi, jnp.int32)


def _transform_pack_tables(h_user, h_item, w):
    rows, d = h_user.shape
    blk2 = 1000
    grid = rows // 2 // blk2

    def body(xu_ref, xi_ref, w_ref, ou_ref, oi_ref):
        xu = xu_ref[...]
        wt = w_ref[...]
        pu = []
        pi = []
        for half in range(2):
            y = lax.dot_general(
                xu[:, half, :], wt,
                dimension_numbers=(((1,), (1,)), ((), ())),
                preferred_element_type=jnp.float32)
            pu.append(_pack_words(y))
            pi.append(_pack_words(xi_ref[:, half, :]))
        ou_ref[...] = lax.concatenate(pu, 1)
        oi_ref[...] = lax.concatenate(pi, 1)

    return pl.pallas_call(
        body,
        grid=(grid,),
        in_specs=[
            pl.BlockSpec((blk2, 2, d), lambda i: (i, 0, 0)),
            pl.BlockSpec((blk2, 2, d), lambda i: (i, 0, 0)),
            pl.BlockSpec((d, d), lambda i: (0, 0)),
        ],
        out_specs=[
            pl.BlockSpec((blk2, d), lambda i: (i, 0)),
            pl.BlockSpec((blk2, d), lambda i: (i, 0)),
        ],
        out_shape=[
            jax.ShapeDtypeStruct((rows // 2, d), jnp.int32),
            jax.ShapeDtypeStruct((rows // 2, d), jnp.int32),
        ],
    )(h_user.reshape(rows // 2, 2, d), h_item.reshape(rows // 2, 2, d), w)


DW = D // 2


def _make_sc_packer(rows_total):
    per_w = rows_total // NW
    rch = 125
    n_chunks = per_w // rch
    mesh = plsc.VectorSubcoreMesh(core_axis_name="c", subcore_axis_name="s")

    @functools.partial(
        pl.kernel,
        mesh=mesh,
        compiler_params=pltpu.CompilerParams(
            needs_layout_passes=False, use_tc_tiling_on_sc=False),
        out_type=[jax.ShapeDtypeStruct((rows_total, DW), jnp.int32),
                  jax.ShapeDtypeStruct((rows_total, DW), jnp.int32)],
        scratch_types=[
            pltpu.VMEM((rch * D,), jnp.float32),
            pltpu.VMEM((rch * D,), jnp.float32),
            pltpu.VMEM((rch * D,), jnp.float32),
            pltpu.VMEM((rch * D,), jnp.float32),
            pltpu.VMEM((rch, DW), jnp.int32),
            pltpu.VMEM((rch, DW), jnp.int32),
            pltpu.VMEM((rch, DW), jnp.int32),
            pltpu.VMEM((rch, DW), jnp.int32),
            pltpu.SemaphoreType.DMA,
            pltpu.SemaphoreType.DMA,
            pltpu.SemaphoreType.DMA,
            pltpu.SemaphoreType.DMA,
            pltpu.SemaphoreType.DMA,
            pltpu.SemaphoreType.DMA,
            pltpu.SemaphoreType.DMA,
            pltpu.SemaphoreType.DMA,
        ],
    )
    def packer(hu_flat, hi_flat, hu_out, hi_out,
               hu_in_a, hu_in_b, hi_in_a, hi_in_b,
               hu_pk_a, hu_pk_b, hi_pk_a, hi_pk_b,
               s_hu_a, s_hu_b, s_hi_a, s_hi_b,
               so_hu_a, so_hu_b, so_hi_a, so_hi_b):
        wid = lax.axis_index("s") * NC + lax.axis_index("c")
        base_row = wid * per_w

        def fire_in(tab, c, buf, sem):
            off = (base_row + c * rch) * D
            pltpu.async_copy(tab.at[pl.ds(off, rch * D)], buf, sem)

        def drain_in(tab, buf, sem):
            pltpu.make_async_copy(tab.at[pl.ds(0, rch * D)], buf, sem).wait()

        def fire_out(out_hbm, c, pk, sem):
            pltpu.async_copy(pk, out_hbm.at[pl.ds(base_row + c * rch, rch)], sem)

        def drain_out(out_hbm, pk, sem):
            pltpu.make_async_copy(pk, out_hbm.at[pl.ds(base_row, rch)], sem).wait()

        def compute(buf, pk):
            def row_body(r, _):
                for k in range(DW // LANES):
                    a = buf[pl.ds(r * D + 2 * k * LANES, LANES)]
                    b = buf[pl.ds(r * D + (2 * k + 1) * LANES, LANES)]
                    pk[r, pl.ds(k * LANES, LANES)] = plsc.bitcast(
                        plsc.pack(a, b, format=plsc.PackFormat.INTERLEAVED),
                        jnp.int32)
                return 0
            lax.fori_loop(0, rch, row_body, 0)

        def step(tab, out_hbm, c, in_buf, in_sem, nxt_buf, nxt_sem,
                 pk, pk_sem):
            drain_in(tab, in_buf, in_sem)
            fire_in(tab, c + 1, nxt_buf, nxt_sem)
            drain_out(out_hbm, pk, pk_sem)
            compute(in_buf, pk)
            fire_out(out_hbm, c, pk, pk_sem)

        fire_in(hu_flat, 0, hu_in_a, s_hu_a)
        fire_in(hi_flat, 0, hi_in_a, s_hi_a)
        fire_out(hu_out, 0, hu_pk_a, so_hu_a)
        fire_out(hu_out, 1, hu_pk_b, so_hu_b)
        fire_out(hi_out, 0, hi_pk_a, so_hi_a)
        fire_out(hi_out, 1, hi_pk_b, so_hi_b)

        def pair_body(p, _):
            c0 = 2 * p
            step(hu_flat, hu_out, c0, hu_in_a, s_hu_a, hu_in_b, s_hu_b,
                 hu_pk_a, so_hu_a)
            step(hi_flat, hi_out, c0, hi_in_a, s_hi_a, hi_in_b, s_hi_b,
                 hi_pk_a, so_hi_a)
            step(hu_flat, hu_out, c0 + 1, hu_in_b, s_hu_b, hu_in_a, s_hu_a,
                 hu_pk_b, so_hu_b)
            step(hi_flat, hi_out, c0 + 1, hi_in_b, s_hi_b, hi_in_a, s_hi_a,
                 hi_pk_b, so_hi_b)
            return 0

        lax.fori_loop(0, (n_chunks - 1) // 2, pair_body, 0)

        for tab, out_hbm, in_buf, in_sem, pk, pk_sem in (
                (hu_flat, hu_out, hu_in_a, s_hu_a, hu_pk_a, so_hu_a),
                (hi_flat, hi_out, hi_in_a, s_hi_a, hi_pk_a, so_hi_a)):
            drain_in(tab, in_buf, in_sem)
            drain_out(out_hbm, pk, pk_sem)
            compute(in_buf, pk)
            fire_out(out_hbm, n_chunks - 1, pk, pk_sem)
            drain_out(out_hbm, pk, pk_sem)
        drain_out(hu_out, hu_pk_b, so_hu_b)
        drain_out(hi_out, hi_pk_b, so_hi_b)

    return packer


def _make_sc_scorer(e_total):
    per_w = e_total // NW
    n_chunks = per_w // CHUNK
    groups = CHUNK // LANES
    mesh = plsc.VectorSubcoreMesh(core_axis_name="c", subcore_axis_name="s")

    @functools.partial(
        pl.kernel,
        mesh=mesh,
        compiler_params=pltpu.CompilerParams(
            needs_layout_passes=False, use_tc_tiling_on_sc=False),
        out_type=jax.ShapeDtypeStruct((e_total,), jnp.float32),
        scratch_types=[
            pltpu.VMEM((per_w,), jnp.int32),
            pltpu.VMEM((per_w,), jnp.int32),
            pltpu.VMEM((CHUNK,), jnp.float32),
            pltpu.VMEM((CHUNK, DW), jnp.int32),
            pltpu.VMEM((CHUNK, DW), jnp.int32),
            pltpu.VMEM((CHUNK, DW), jnp.int32),
            pltpu.VMEM((CHUNK, DW), jnp.int32),
            pltpu.SemaphoreType.DMA,
            pltpu.SemaphoreType.DMA,
        ],
    )
    def scorer(hu_t, hi, src_hbm, dst_hbm, out_hbm,
               sidx_v, didx_v, out_v, srows_a, drows_a, srows_b, drows_b,
               sem_a, sem_b):
        wid = lax.axis_index("s") * NC + lax.axis_index("c")
        base = wid * per_w
        pltpu.sync_copy(src_hbm.at[pl.ds(base, per_w)], sidx_v)
        pltpu.sync_copy(dst_hbm.at[pl.ds(base, per_w)], didx_v)

        def fire(c, s_buf, d_buf, sem):
            off = c * CHUNK
            pltpu.async_copy(hu_t.at[sidx_v.at[pl.ds(off, CHUNK)]], s_buf, sem)
            pltpu.async_copy(hi.at[didx_v.at[pl.ds(off, CHUNK)]], d_buf, sem)

        def drain(s_buf, d_buf, sem):
            pltpu.make_async_copy(hu_t.at[sidx_v.at[pl.ds(0, CHUNK)]], s_buf, sem).wait()
            pltpu.make_async_copy(hi.at[didx_v.at[pl.ds(0, CHUNK)]], d_buf, sem).wait()

        lane_iota = lax.iota(jnp.int32, LANES)

        def compute_chunk(c, s_ref, d_ref):
            def group_body(g, _):
                res = jnp.zeros((LANES,), jnp.float32)
                for j in range(LANES):
                    accs = []
                    for k in range(DW // LANES):
                        sw = s_ref[g * LANES + j, pl.ds(k * LANES, LANES)]
                        dw = d_ref[g * LANES + j, pl.ds(k * LANES, LANES)]
                        prod = (plsc.bitcast(sw, jnp.bfloat16)
                                * plsc.bitcast(dw, jnp.bfloat16))
                        p0, p1 = plsc.unpack(
                            prod, format=plsc.PackFormat.INTERLEAVED)
                        accs.append(p0 + p1)
                    acc = (accs[0] + accs[1]) + (accs[2] + accs[3])
                    res = jnp.where(lane_iota == j, jnp.sum(acc), res)
                out_v[pl.ds(g * LANES, LANES)] = res
                return 0
            lax.fori_loop(0, groups, group_body, 0)
            pltpu.sync_copy(out_v, out_hbm.at[pl.ds(base + c * CHUNK, CHUNK)])

        assert n_chunks % 2 == 1
        fire(0, srows_a, drows_a, sem_a)

        def pair_body(p, _):
            c0 = 2 * p
            drain(srows_a, drows_a, sem_a)
            fire(c0 + 1, srows_b, drows_b, sem_b)
            compute_chunk(c0, srows_a, drows_a)
            drain(srows_b, drows_b, sem_b)
            fire(c0 + 2, srows_a, drows_a, sem_a)
            compute_chunk(c0 + 1, srows_b, drows_b)
            return 0

        lax.fori_loop(0, (n_chunks - 1) // 2, pair_body, 0)
        drain(srows_a, drows_a, sem_a)
        compute_chunk(n_chunks - 1, srows_a, drows_a)

    return scorer


def kernel(h_user, h_item, W, src_idx, dst_idx):
    rows = h_user.shape[0]
    hu_p2, hi_p2 = _transform_pack_tables(h_user, h_item, W)
    scorer = _make_sc_scorer(src_idx.shape[0])
    return scorer(hu_p2.reshape(rows, DW), hi_p2.reshape(rows, DW),
                  src_idx, dst_idx)


# speedup vs baseline: 1.8144x; 1.0015x over previous
import functools

import jax
import jax.numpy as jnp
from jax import lax
from jax.experimental import pallas as pl
from jax.experimental.pallas import tpu as pltpu
from jax.experimental.pallas import tpu_sc as plsc

D = 128
NC = 2
NS = 16
NW = NC * NS
CHUNK = 400
LANES = 16


def _pack_words(y):
    b = lax.bitcast_convert_type(y, jnp.uint32) + jnp.uint32(0x8000)
    lo = b[:, :DW] >> jnp.uint32(16)
    hi = b[:, DW:] & jnp.uint32(0xFFFF0000)
    return lax.bitcast_convert_type(lo | hi, jnp.int32)


def _transform_pack_tables(h_user, h_item, w):
    rows, d = h_user.shape
    blk2 = 2000
    grid = rows // 2 // blk2

    def body(xu_ref, xi_ref, w_ref, ou_ref, oi_ref):
        xu = xu_ref[...]
        wt = w_ref[...]
        pu = []
        pi = []
        for half in range(2):
            y = lax.dot_general(
                xu[:, half, :], wt,
                dimension_numbers=(((1,), (1,)), ((), ())),
                preferred_element_type=jnp.float32)
            pu.append(_pack_words(y))
            pi.append(_pack_words(xi_ref[:, half, :]))
        ou_ref[...] = lax.concatenate(pu, 1)
        oi_ref[...] = lax.concatenate(pi, 1)

    return pl.pallas_call(
        body,
        grid=(grid,),
        in_specs=[
            pl.BlockSpec((blk2, 2, d), lambda i: (i, 0, 0)),
            pl.BlockSpec((blk2, 2, d), lambda i: (i, 0, 0)),
            pl.BlockSpec((d, d), lambda i: (0, 0)),
        ],
        out_specs=[
            pl.BlockSpec((blk2, d), lambda i: (i, 0)),
            pl.BlockSpec((blk2, d), lambda i: (i, 0)),
        ],
        out_shape=[
            jax.ShapeDtypeStruct((rows // 2, d), jnp.int32),
            jax.ShapeDtypeStruct((rows // 2, d), jnp.int32),
        ],
    )(h_user.reshape(rows // 2, 2, d), h_item.reshape(rows // 2, 2, d), w)


DW = D // 2


def _make_sc_packer(rows_total):
    per_w = rows_total // NW
    rch = 125
    n_chunks = per_w // rch
    mesh = plsc.VectorSubcoreMesh(core_axis_name="c", subcore_axis_name="s")

    @functools.partial(
        pl.kernel,
        mesh=mesh,
        compiler_params=pltpu.CompilerParams(
            needs_layout_passes=False, use_tc_tiling_on_sc=False),
        out_type=[jax.ShapeDtypeStruct((rows_total, DW), jnp.int32),
                  jax.ShapeDtypeStruct((rows_total, DW), jnp.int32)],
        scratch_types=[
            pltpu.VMEM((rch * D,), jnp.float32),
            pltpu.VMEM((rch * D,), jnp.float32),
            pltpu.VMEM((rch * D,), jnp.float32),
            pltpu.VMEM((rch * D,), jnp.float32),
            pltpu.VMEM((rch, DW), jnp.int32),
            pltpu.VMEM((rch, DW), jnp.int32),
            pltpu.VMEM((rch, DW), jnp.int32),
            pltpu.VMEM((rch, DW), jnp.int32),
            pltpu.SemaphoreType.DMA,
            pltpu.SemaphoreType.DMA,
            pltpu.SemaphoreType.DMA,
            pltpu.SemaphoreType.DMA,
            pltpu.SemaphoreType.DMA,
            pltpu.SemaphoreType.DMA,
            pltpu.SemaphoreType.DMA,
            pltpu.SemaphoreType.DMA,
        ],
    )
    def packer(hu_flat, hi_flat, hu_out, hi_out,
               hu_in_a, hu_in_b, hi_in_a, hi_in_b,
               hu_pk_a, hu_pk_b, hi_pk_a, hi_pk_b,
               s_hu_a, s_hu_b, s_hi_a, s_hi_b,
               so_hu_a, so_hu_b, so_hi_a, so_hi_b):
        wid = lax.axis_index("s") * NC + lax.axis_index("c")
        base_row = wid * per_w

        def fire_in(tab, c, buf, sem):
            off = (base_row + c * rch) * D
            pltpu.async_copy(tab.at[pl.ds(off, rch * D)], buf, sem)

        def drain_in(tab, buf, sem):
            pltpu.make_async_copy(tab.at[pl.ds(0, rch * D)], buf, sem).wait()

        def fire_out(out_hbm, c, pk, sem):
            pltpu.async_copy(pk, out_hbm.at[pl.ds(base_row + c * rch, rch)], sem)

        def drain_out(out_hbm, pk, sem):
            pltpu.make_async_copy(pk, out_hbm.at[pl.ds(base_row, rch)], sem).wait()

        def compute(buf, pk):
            def row_body(r, _):
                for k in range(DW // LANES):
                    a = buf[pl.ds(r * D + 2 * k * LANES, LANES)]
                    b = buf[pl.ds(r * D + (2 * k + 1) * LANES, LANES)]
                    pk[r, pl.ds(k * LANES, LANES)] = plsc.bitcast(
                        plsc.pack(a, b, format=plsc.PackFormat.INTERLEAVED),
                        jnp.int32)
                return 0
            lax.fori_loop(0, rch, row_body, 0)

        def step(tab, out_hbm, c, in_buf, in_sem, nxt_buf, nxt_sem,
                 pk, pk_sem):
            drain_in(tab, in_buf, in_sem)
            fire_in(tab, c + 1, nxt_buf, nxt_sem)
            drain_out(out_hbm, pk, pk_sem)
            compute(in_buf, pk)
            fire_out(out_hbm, c, pk, pk_sem)

        fire_in(hu_flat, 0, hu_in_a, s_hu_a)
        fire_in(hi_flat, 0, hi_in_a, s_hi_a)
        fire_out(hu_out, 0, hu_pk_a, so_hu_a)
        fire_out(hu_out, 1, hu_pk_b, so_hu_b)
        fire_out(hi_out, 0, hi_pk_a, so_hi_a)
        fire_out(hi_out, 1, hi_pk_b, so_hi_b)

        def pair_body(p, _):
            c0 = 2 * p
            step(hu_flat, hu_out, c0, hu_in_a, s_hu_a, hu_in_b, s_hu_b,
                 hu_pk_a, so_hu_a)
            step(hi_flat, hi_out, c0, hi_in_a, s_hi_a, hi_in_b, s_hi_b,
                 hi_pk_a, so_hi_a)
            step(hu_flat, hu_out, c0 + 1, hu_in_b, s_hu_b, hu_in_a, s_hu_a,
                 hu_pk_b, so_hu_b)
            step(hi_flat, hi_out, c0 + 1, hi_in_b, s_hi_b, hi_in_a, s_hi_a,
                 hi_pk_b, so_hi_b)
            return 0

        lax.fori_loop(0, (n_chunks - 1) // 2, pair_body, 0)

        for tab, out_hbm, in_buf, in_sem, pk, pk_sem in (
                (hu_flat, hu_out, hu_in_a, s_hu_a, hu_pk_a, so_hu_a),
                (hi_flat, hi_out, hi_in_a, s_hi_a, hi_pk_a, so_hi_a)):
            drain_in(tab, in_buf, in_sem)
            drain_out(out_hbm, pk, pk_sem)
            compute(in_buf, pk)
            fire_out(out_hbm, n_chunks - 1, pk, pk_sem)
            drain_out(out_hbm, pk, pk_sem)
        drain_out(hu_out, hu_pk_b, so_hu_b)
        drain_out(hi_out, hi_pk_b, so_hi_b)

    return packer


def _make_sc_scorer(e_total):
    per_w = e_total // NW
    n_chunks = per_w // CHUNK
    groups = CHUNK // LANES
    mesh = plsc.VectorSubcoreMesh(core_axis_name="c", subcore_axis_name="s")

    @functools.partial(
        pl.kernel,
        mesh=mesh,
        compiler_params=pltpu.CompilerParams(
            needs_layout_passes=False, use_tc_tiling_on_sc=False),
        out_type=jax.ShapeDtypeStruct((e_total,), jnp.float32),
        scratch_types=[
            pltpu.VMEM((per_w,), jnp.int32),
            pltpu.VMEM((per_w,), jnp.int32),
            pltpu.VMEM((CHUNK,), jnp.float32),
            pltpu.VMEM((CHUNK, DW), jnp.int32),
            pltpu.VMEM((CHUNK, DW), jnp.int32),
            pltpu.VMEM((CHUNK, DW), jnp.int32),
            pltpu.VMEM((CHUNK, DW), jnp.int32),
            pltpu.SemaphoreType.DMA,
            pltpu.SemaphoreType.DMA,
        ],
    )
    def scorer(hu_t, hi, src_hbm, dst_hbm, out_hbm,
               sidx_v, didx_v, out_v, srows_a, drows_a, srows_b, drows_b,
               sem_a, sem_b):
        wid = lax.axis_index("s") * NC + lax.axis_index("c")
        base = wid * per_w
        pltpu.sync_copy(src_hbm.at[pl.ds(base, per_w)], sidx_v)
        pltpu.sync_copy(dst_hbm.at[pl.ds(base, per_w)], didx_v)

        def fire(c, s_buf, d_buf, sem):
            off = c * CHUNK
            pltpu.async_copy(hu_t.at[sidx_v.at[pl.ds(off, CHUNK)]], s_buf, sem)
            pltpu.async_copy(hi.at[didx_v.at[pl.ds(off, CHUNK)]], d_buf, sem)

        def drain(s_buf, d_buf, sem):
            pltpu.make_async_copy(hu_t.at[sidx_v.at[pl.ds(0, CHUNK)]], s_buf, sem).wait()
            pltpu.make_async_copy(hi.at[didx_v.at[pl.ds(0, CHUNK)]], d_buf, sem).wait()

        lane_iota = lax.iota(jnp.int32, LANES)

        def compute_chunk(c, s_ref, d_ref):
            def group_body(g, _):
                res = jnp.zeros((LANES,), jnp.float32)
                for j in range(LANES):
                    accs = []
                    for k in range(DW // LANES):
                        sw = s_ref[g * LANES + j, pl.ds(k * LANES, LANES)]
                        dw = d_ref[g * LANES + j, pl.ds(k * LANES, LANES)]
                        prod = (plsc.bitcast(sw, jnp.bfloat16)
                                * plsc.bitcast(dw, jnp.bfloat16))
                        p0, p1 = plsc.unpack(
                            prod, format=plsc.PackFormat.INTERLEAVED)
                        accs.append(p0 + p1)
                    acc = (accs[0] + accs[1]) + (accs[2] + accs[3])
                    res = jnp.where(lane_iota == j, jnp.sum(acc), res)
                out_v[pl.ds(g * LANES, LANES)] = res
                return 0
            lax.fori_loop(0, groups, group_body, 0)
            pltpu.sync_copy(out_v, out_hbm.at[pl.ds(base + c * CHUNK, CHUNK)])

        assert n_chunks % 2 == 1
        fire(0, srows_a, drows_a, sem_a)

        def pair_body(p, _):
            c0 = 2 * p
            drain(srows_a, drows_a, sem_a)
            fire(c0 + 1, srows_b, drows_b, sem_b)
            compute_chunk(c0, srows_a, drows_a)
            drain(srows_b, drows_b, sem_b)
            fire(c0 + 2, srows_a, drows_a, sem_a)
            compute_chunk(c0 + 1, srows_b, drows_b)
            return 0

        lax.fori_loop(0, (n_chunks - 1) // 2, pair_body, 0)
        drain(srows_a, drows_a, sem_a)
        compute_chunk(n_chunks - 1, srows_a, drows_a)

    return scorer


def kernel(h_user, h_item, W, src_idx, dst_idx):
    rows = h_user.shape[0]
    hu_p2, hi_p2 = _transform_pack_tables(h_user, h_item, W)
    scorer = _make_sc_scorer(src_idx.shape[0])
    return scorer(hu_p2.reshape(rows, DW), hi_p2.reshape(rows, DW),
                  src_idx, dst_idx)
